# baseline (speedup 1.0000x reference)
import functools

import jax
import jax.numpy as jnp
from jax import lax
from jax.experimental import pallas as pl
from jax.experimental.pallas import tpu as pltpu
from jax.experimental.pallas import tpu_sc as plsc

NUM_COLS = 100000
COL_DIM = 64
NUM_OPS = 6
OP_DIM = 8
B = 16384
L = 20
N = B * L
OUT_DIM = COL_DIM + OP_DIM + COL_DIM + 2

NW = 32
ROWS_PER_W = N // NW
C = 512
NCHUNK = ROWS_PER_W // C
NB = C // 128
ZROW = NUM_COLS


def _body(tab_hbm, opemb_hbm, col1_hbm, c2n_hbm, op_hbm, join_hbm, tail_hbm, out_hbm,
          idx1_v, c2n_v, idx2_v, opidx_v, join_v,
          buf1_v, buf2_v, bufop_v, tail_v, sem):
    wid = lax.axis_index("s") * 2 + lax.axis_index("c")

    def chunk(k, _):
        rowbase = wid * ROWS_PER_W + k * C
        blkbase = wid * (ROWS_PER_W // 128) + k * NB

        with jax.named_scope("in_copies"):
            pltpu.sync_copy(col1_hbm.at[pl.ds(blkbase, NB)], idx1_v)
            pltpu.sync_copy(c2n_hbm.at[pl.ds(blkbase, NB)], c2n_v)
            pltpu.sync_copy(op_hbm.at[pl.ds(blkbase, NB)], opidx_v)
            pltpu.sync_copy(join_hbm.at[pl.ds(blkbase, NB)], join_v)
            pltpu.sync_copy(tail_hbm.at[pl.ds(rowbase, C)], tail_v)

        with jax.named_scope("idx_compute"):
            for j in range(C // 16):
                r2, off = (j * 16) // 128, (j * 16) % 128
                g = join_v[r2, pl.ds(off, 16)]
                c2 = c2n_v[r2, pl.ds(off, 16)]
                idx2_v[r2, pl.ds(off, 16)] = jnp.where(g != 0, c2, ZROW)

        with jax.named_scope("gathers"):
            cps = []
            for s in range(NB):
                cps.append(pltpu.async_copy(
                    tab_hbm.at[idx1_v.at[s]], buf1_v.at[pl.ds(s * 128, 128)], sem))
                cps.append(pltpu.async_copy(
                    tab_hbm.at[idx2_v.at[s]], buf2_v.at[pl.ds(s * 128, 128)], sem))
            for cp in cps:
                cp.wait()

        with jax.named_scope("out_copies"):
            pltpu.sync_copy(buf1_v, out_hbm.at[pl.ds(rowbase, C), pl.ds(0, COL_DIM)])
            pltpu.sync_copy(bufop_v, out_hbm.at[pl.ds(rowbase, C), pl.ds(COL_DIM, OP_DIM)])
            pltpu.sync_copy(buf2_v, out_hbm.at[pl.ds(rowbase, C), pl.ds(COL_DIM + OP_DIM, COL_DIM)])
            pltpu.sync_copy(tail_v, out_hbm.at[pl.ds(rowbase, C), pl.ds(OUT_DIM - 2, 2)])
        return ()

    lax.fori_loop(0, NCHUNK, chunk, ())


@jax.jit
def _encode(tab, opemb, col1, c2n, opi, join, tail):
    mesh = plsc.VectorSubcoreMesh(core_axis_name="c", subcore_axis_name="s")
    return pl.kernel(
        _body,
        out_type=jax.ShapeDtypeStruct((N, OUT_DIM), jnp.float32),
        mesh=mesh,
        compiler_params=pltpu.CompilerParams(use_tc_tiling_on_sc=False),
        scratch_types=[
            pltpu.VMEM((NB, 128), jnp.int32),
            pltpu.VMEM((NB, 128), jnp.int32),
            pltpu.VMEM((NB, 128), jnp.int32),
            pltpu.VMEM((NB, 128), jnp.int32),
            pltpu.VMEM((NB, 128), jnp.int32),
            pltpu.VMEM((C, COL_DIM), jnp.float32),
            pltpu.VMEM((C, COL_DIM), jnp.float32),
            pltpu.VMEM((C, OP_DIM), jnp.float32),
            pltpu.VMEM((C, 2), jnp.float32),
            pltpu.SemaphoreType.DMA,
        ],
    )(tab, opemb, col1, c2n, opi, join, tail)


def kernel(col1, op, col2_or_num, is_join, col_emb, op_emb):
    as_blocks = lambda a: a.reshape(-1).astype(jnp.int32).reshape(N // 128, 128)
    tab = jnp.concatenate(
        [col_emb.astype(jnp.float32), jnp.zeros((8, COL_DIM), jnp.float32)], axis=0)
    gate = is_join.reshape(-1).astype(jnp.float32)
    num = col2_or_num.reshape(-1).astype(jnp.float32) * (1.0 - gate)
    tail = jnp.stack([num, gate], axis=-1)
    out = _encode(tab, op_emb.astype(jnp.float32), as_blocks(col1),
                  as_blocks(col2_or_num), as_blocks(op), as_blocks(is_join), tail)
    return out.reshape(B, L, OUT_DIM)


# speedup vs baseline: 1.0011x; 1.0011x over previous
import jax
import jax.numpy as jnp
from jax import lax
from jax.experimental import pallas as pl
from jax.experimental.pallas import tpu as pltpu
from jax.experimental.pallas import tpu_sc as plsc

NUM_COLS = 100000
COL_DIM = 64
NUM_OPS = 6
OP_DIM = 8
B = 16384
L = 20
N = B * L
OUT_DIM = COL_DIM + OP_DIM + COL_DIM + 2

NW = 32
ROWS_PER_W = N // NW
C = 512
NCHUNK = ROWS_PER_W // C
NB = C // 128
ZROW = NUM_COLS


def _body(tab_hbm, opemb_hbm, col1_hbm, c2n_hbm, op_hbm, join_hbm, tail_hbm, out_hbm,
          idx1_v, c2n_v, idx2_v, opidx_v, join_v,
          buf1_v, buf2_v, bufop_v, tail_v, sem):
    wid = lax.axis_index("s") * 2 + lax.axis_index("c")

    def chunk(k, _):
        rowbase = wid * ROWS_PER_W + k * C
        blkbase = wid * (ROWS_PER_W // 128) + k * NB

        with jax.named_scope("in_copies"):
            pltpu.sync_copy(col1_hbm.at[pl.ds(blkbase, NB)], idx1_v)
            pltpu.sync_copy(c2n_hbm.at[pl.ds(blkbase, NB)], c2n_v)
            pltpu.sync_copy(op_hbm.at[pl.ds(blkbase, NB)], opidx_v)
            pltpu.sync_copy(join_hbm.at[pl.ds(blkbase, NB)], join_v)
            pltpu.sync_copy(tail_hbm.at[pl.ds(rowbase, C)], tail_v)

        with jax.named_scope("idx_compute"):
            for j in range(C // 16):
                r2, off = (j * 16) // 128, (j * 16) % 128
                g = join_v[r2, pl.ds(off, 16)]
                c2 = c2n_v[r2, pl.ds(off, 16)]
                idx2_v[r2, pl.ds(off, 16)] = jnp.where(g != 0, c2, ZROW)

        with jax.named_scope("gathers"):
            cps = []
            for s in range(NB):
                cps.append(pltpu.async_copy(
                    tab_hbm.at[idx1_v.at[s]], buf1_v.at[pl.ds(s * 128, 128)], sem))
                cps.append(pltpu.async_copy(
                    tab_hbm.at[idx2_v.at[s]], buf2_v.at[pl.ds(s * 128, 128)], sem))
                cps.append(pltpu.async_copy(
                    opemb_hbm.at[opidx_v.at[s]], bufop_v.at[pl.ds(s * 128, 128)], sem))
            for cp in cps:
                cp.wait()

        with jax.named_scope("out_copies"):
            pltpu.sync_copy(buf1_v, out_hbm.at[pl.ds(rowbase, C), pl.ds(0, COL_DIM)])
            pltpu.sync_copy(bufop_v, out_hbm.at[pl.ds(rowbase, C), pl.ds(COL_DIM, OP_DIM)])
            pltpu.sync_copy(buf2_v, out_hbm.at[pl.ds(rowbase, C), pl.ds(COL_DIM + OP_DIM, COL_DIM)])
            pltpu.sync_copy(tail_v, out_hbm.at[pl.ds(rowbase, C), pl.ds(OUT_DIM - 2, 2)])
        return ()

    lax.fori_loop(0, NCHUNK, chunk, ())


@jax.jit
def _encode(tab, opemb, col1, c2n, opi, join, tail):
    mesh = plsc.VectorSubcoreMesh(core_axis_name="c", subcore_axis_name="s")
    return pl.kernel(
        _body,
        out_type=jax.ShapeDtypeStruct((N, OUT_DIM), jnp.float32),
        mesh=mesh,
        compiler_params=pltpu.CompilerParams(use_tc_tiling_on_sc=False),
        scratch_types=[
            pltpu.VMEM((NB, 128), jnp.int32),
            pltpu.VMEM((NB, 128), jnp.int32),
            pltpu.VMEM((NB, 128), jnp.int32),
            pltpu.VMEM((NB, 128), jnp.int32),
            pltpu.VMEM((NB, 128), jnp.int32),
            pltpu.VMEM((C, COL_DIM), jnp.float32),
            pltpu.VMEM((C, COL_DIM), jnp.float32),
            pltpu.VMEM((C, OP_DIM), jnp.float32),
            pltpu.VMEM((C, 2), jnp.float32),
            pltpu.SemaphoreType.DMA,
        ],
    )(tab, opemb, col1, c2n, opi, join, tail)


def kernel(col1, op, col2_or_num, is_join, col_emb, op_emb):
    as_blocks = lambda a: a.reshape(-1).astype(jnp.int32).reshape(N // 128, 128)
    tab = jnp.concatenate(
        [col_emb.astype(jnp.float32), jnp.zeros((8, COL_DIM), jnp.float32)], axis=0)
    gate = is_join.reshape(-1).astype(jnp.float32)
    num = col2_or_num.reshape(-1).astype(jnp.float32) * (1.0 - gate)
    tail = jnp.stack([num, gate], axis=-1)
    out = _encode(tab, op_emb.astype(jnp.float32), as_blocks(col1),
                  as_blocks(col2_or_num), as_blocks(op), as_blocks(is_join), tail)
    return out.reshape(B, L, OUT_DIM)
